# baseline (speedup 1.0000x reference)
import functools

import jax
import jax.numpy as jnp
from jax import lax
from jax.experimental import pallas as pl
from jax.experimental.pallas import tpu as pltpu
from jax.experimental.pallas import tpu_sc as plsc

NH = 12
HD = 64

_I = False


def _rms(x, scale):
    return x * lax.rsqrt(jnp.mean(x * x, axis=-1, keepdims=True) + 1e-6) * scale


def _emb_gather(table, idx_flat):
    T = idx_flat.shape[0]
    V, D = table.shape
    NC, NS = 2, 16
    NW = NC * NS
    b_per_w = T // NW
    mesh = plsc.VectorSubcoreMesh(core_axis_name="c", subcore_axis_name="s")

    @functools.partial(
        pl.kernel, mesh=mesh,
        out_type=jax.ShapeDtypeStruct((T, D), jnp.float32),
        scratch_types=[
            pltpu.VMEM((b_per_w,), jnp.int32),
            pltpu.VMEM((b_per_w, D), jnp.float32),
            pltpu.SemaphoreType.DMA,
        ],
    )
    def gk(table_hbm, idx_hbm, out_hbm, idx_v, rows_v, sem):
        wid = lax.axis_index("s") * NC + lax.axis_index("c")
        base = wid * b_per_w
        pltpu.sync_copy(idx_hbm.at[pl.ds(base, b_per_w)], idx_v)
        pltpu.async_copy(table_hbm.at[idx_v], rows_v, sem).wait()
        pltpu.sync_copy(rows_v, out_hbm.at[pl.ds(base, b_per_w)])

    return gk(table, idx_flat)


def _qkv_body(x_ref, sc_ref, c32_ref, s32_ref, ec_ref, es_ref,
              wqk_ref, wqks_ref,
              wv_ref, bqk_ref, bv_ref, q_ref, k_ref, v_ref,
              c_s, s_s, *, blk):
    h = pl.program_id(1)
    dn0 = (((1,), (0,)), ((), ()))

    @pl.when(h == 0)
    def _():
        c_s[...] = lax.dot_general(c32_ref[...], ec_ref[...], dn0,
                                   preferred_element_type=jnp.float32)
        s_s[...] = lax.dot_general(s32_ref[...], es_ref[...], dn0,
                                   preferred_element_type=jnp.float32)

    xn = _rms(x_ref[...], sc_ref[...])
    a = xn * c_s[...]
    b = xn * s_s[...]
    dn = (((1,), (0,)), ((), ()))

    def mm(u, w_ref):
        return lax.dot_general(u, w_ref[0], dn,
                               preferred_element_type=jnp.float32)

    qk = mm(a, wqk_ref) + mm(b, wqks_ref) + bqk_ref[0]
    q_ref[0] = qk[:, :HD]
    k_ref[0] = qk[:, HD:]
    v_ref[0] = mm(xn, wv_ref) + bv_ref[0]


def _qkv(x, scale, c32, s32, ec, es, wqk4, wqks4, wvT3, bqk3, bv3,
         blk=512):
    T, D = x.shape
    HH = c32.shape[1]
    out = jax.ShapeDtypeStruct((NH, T, HD), jnp.float32)
    wspec = pl.BlockSpec((1, D, 2 * HD), lambda i, h: (h, 0, 0))
    return pl.pallas_call(
        functools.partial(_qkv_body, blk=blk),
        grid=(T // blk, NH),
        in_specs=[
            pl.BlockSpec((blk, D), lambda i, h: (i, 0)),
            pl.BlockSpec((D,), lambda i, h: (0,)),
            pl.BlockSpec((blk, HH), lambda i, h: (i, 0)),
            pl.BlockSpec((blk, HH), lambda i, h: (i, 0)),
            pl.BlockSpec((HH, D), lambda i, h: (0, 0)),
            pl.BlockSpec((HH, D), lambda i, h: (0, 0)),
            wspec, wspec,
            pl.BlockSpec((1, D, HD), lambda i, h: (h, 0, 0)),
            pl.BlockSpec((1, 1, 2 * HD), lambda i, h: (h, 0, 0)),
            pl.BlockSpec((1, 1, HD), lambda i, h: (h, 0, 0)),
        ],
        out_specs=[pl.BlockSpec((1, blk, HD), lambda i, h: (h, i, 0))] * 3,
        out_shape=[out, out, out],
        scratch_shapes=[pltpu.VMEM((blk, D), jnp.float32),
                        pltpu.VMEM((blk, D), jnp.float32)],
        interpret=_I,
    )(x, scale, c32, s32, ec, es, wqk4, wqks4, wvT3, bqk3, bv3)


def _attn_body(q_ref, k_ref, v_ref, o_ref, *, half, bq, win):
    qb = pl.program_id(1)
    q = q_ref[0]
    ntile = half // bq
    start = jnp.maximum(qb - ntile, 0) * bq
    kw = k_ref[0, pl.ds(start, win), :]
    vw = v_ref[0, pl.ds(start, win), :]
    dn = (((1,), (1,)), ((), ()))
    s = lax.dot_general(q, kw, dn, preferred_element_type=jnp.float32) * (HD ** -0.5)
    rows = qb * bq + lax.broadcasted_iota(jnp.int32, (bq, win), 0)
    cols = start + lax.broadcasted_iota(jnp.int32, (bq, win), 1)
    bad = (cols > rows) | (cols <= rows - half)
    s = jnp.where(bad, -jnp.inf, s)
    m = jnp.max(s, axis=-1, keepdims=True)
    p = jnp.exp(s - m)
    denom = jnp.sum(p, axis=-1, keepdims=True)
    o = lax.dot_general(p, vw, (((1,), (0,)), ((), ())),
                        preferred_element_type=jnp.float32)
    o_ref[0] = o / denom


def _attention(q3, k3, v3, half, bq=256):
    NHl, T, HDl = q3.shape
    win = half + bq
    body = functools.partial(_attn_body, half=half, bq=bq, win=win)
    return pl.pallas_call(
        body,
        grid=(NHl, T // bq),
        in_specs=[
            pl.BlockSpec((1, bq, HDl), lambda h, i: (h, i, 0)),
            pl.BlockSpec((1, T, HDl), lambda h, i: (h, 0, 0)),
            pl.BlockSpec((1, T, HDl), lambda h, i: (h, 0, 0)),
        ],
        out_specs=pl.BlockSpec((1, bq, HDl), lambda h, i: (h, i, 0)),
        out_shape=jax.ShapeDtypeStruct((NHl, T, HDl), jnp.float32),
        interpret=_I,
    )(q3, k3, v3)


def _postattn_body(ao_ref, wo_ref, bo_ref, sc_ref, rw_ref, rb_ref,
                   xa_ref, g_ref):
    E = rw_ref.shape[0]
    dn = (((1,), (1,)), ((), ()))
    xa = bo_ref[...] + jnp.zeros((ao_ref.shape[1], wo_ref.shape[2]),
                                 jnp.float32)
    for h in range(ao_ref.shape[0]):
        xa = xa + lax.dot_general(ao_ref[h], wo_ref[h],
                                  (((1,), (0,)), ((), ())),
                                  preferred_element_type=jnp.float32)
    xa_ref[...] = xa
    hn = _rms(xa, sc_ref[...])
    lg = (lax.dot_general(hn, rw_ref[...], dn,
                          preferred_element_type=jnp.float32) + rb_ref[0])
    col = lax.broadcasted_iota(jnp.int32, lg.shape, 1)
    m1 = jnp.max(lg, axis=-1, keepdims=True)
    i1 = jnp.min(jnp.where(lg == m1, col, E), axis=-1, keepdims=True)
    lg2 = jnp.where(col == i1, -jnp.inf, lg)
    m2 = jnp.max(lg2, axis=-1, keepdims=True)
    i2 = jnp.min(jnp.where(lg2 == m2, col, E), axis=-1, keepdims=True)
    keep = (col == i1) | (col == i2)
    sp = jnp.where(keep, lg, -jnp.inf)
    p = jnp.exp(sp - m1)
    g_ref[...] = p / jnp.sum(p, axis=-1, keepdims=True)


def _postattn(ao3, woT3, bo, scale, rw, rb2, blk=256):
    _, T, _ = ao3.shape
    D = woT3.shape[2]
    E = rw.shape[0]
    return pl.pallas_call(
        _postattn_body,
        grid=(T // blk,),
        in_specs=[
            pl.BlockSpec((NH, blk, HD), lambda i: (0, i, 0)),
            pl.BlockSpec((NH, HD, D), lambda i: (0, 0, 0)),
            pl.BlockSpec((D,), lambda i: (0,)),
            pl.BlockSpec((D,), lambda i: (0,)),
            pl.BlockSpec((E, D), lambda i: (0, 0)),
            pl.BlockSpec((1, E), lambda i: (0, 0)),
        ],
        out_specs=[
            pl.BlockSpec((blk, D), lambda i: (i, 0)),
            pl.BlockSpec((blk, E), lambda i: (i, 0)),
        ],
        out_shape=[
            jax.ShapeDtypeStruct((T, D), jnp.float32),
            jax.ShapeDtypeStruct((T, E), jnp.float32),
        ],
        interpret=_I,
    )(ao3, woT3, bo, scale, rw, rb2)


def _moe_body(xa_ref, g_ref, sc_ref, w1_ref, b1_ref, w2_ref, b2_ref, o_ref,
              hn_s):
    e = pl.program_id(0)
    f = pl.program_id(1)
    first = (e == 0) & (f == 0)

    @pl.when(first)
    def _():
        hn_s[...] = _rms(xa_ref[...], sc_ref[...])
        o_ref[...] = xa_ref[...]

    h = lax.dot_general(hn_s[...], w1_ref[0], (((1,), (0,)), ((), ())),
                        preferred_element_type=jnp.float32) + b1_ref[0, 0]
    h = h * jax.nn.sigmoid(h)
    part = lax.dot_general(h, w2_ref[0], (((1,), (0,)), ((), ())),
                           preferred_element_type=jnp.float32)
    g = g_ref[...]
    col = lax.broadcasted_iota(jnp.int32, g.shape, 1)
    ge = jnp.sum(jnp.where(col == e, g, 0.0), axis=-1, keepdims=True)
    contrib = part * ge
    contrib = contrib + jnp.where(f == 0, 1.0, 0.0) * (ge * b2_ref[0, 0])
    o_ref[...] = o_ref[...] + contrib


def _moe(xa, g, scale, W1, b1, W2, b2, bf=768):
    T, D = xa.shape
    E, _, FF = W1.shape
    return pl.pallas_call(
        _moe_body,
        grid=(E, FF // bf),
        in_specs=[
            pl.BlockSpec((T, D), lambda e, f: (0, 0)),
            pl.BlockSpec((T, E), lambda e, f: (0, 0)),
            pl.BlockSpec((D,), lambda e, f: (0,)),
            pl.BlockSpec((1, D, bf), lambda e, f: (e, 0, f)),
            pl.BlockSpec((1, 1, bf), lambda e, f: (e, 0, f)),
            pl.BlockSpec((1, bf, D), lambda e, f: (e, f, 0)),
            pl.BlockSpec((1, 1, D), lambda e, f: (e, 0, 0)),
        ],
        out_specs=pl.BlockSpec((T, D), lambda e, f: (0, 0)),
        out_shape=jax.ShapeDtypeStruct((T, D), jnp.float32),
        scratch_shapes=[pltpu.VMEM((T, D), jnp.float32)],
        interpret=_I,
    )(xa, g, scale, W1, b1.reshape(E, 1, FF), W2, b2.reshape(E, 1, D))


def _lm_body(x_ref, sc_ref, w_ref, b_ref, o_ref, xn_s):
    @pl.when(pl.program_id(0) == 0)
    def _():
        xn_s[...] = _rms(x_ref[...], sc_ref[...])

    o_ref[...] = (lax.dot_general(xn_s[...], w_ref[...],
                                  (((1,), (1,)), ((), ())),
                                  preferred_element_type=jnp.float32)
                  + b_ref[0])


def _lm_head(x2, scale, lm_w, lm_b2, bv=1024):
    T, D = x2.shape
    Vm = lm_w.shape[0]
    return pl.pallas_call(
        _lm_body,
        grid=(pl.cdiv(Vm, bv),),
        in_specs=[
            pl.BlockSpec((T, D), lambda i: (0, 0)),
            pl.BlockSpec((D,), lambda i: (0,)),
            pl.BlockSpec((bv, D), lambda i: (i, 0)),
            pl.BlockSpec((1, bv), lambda i: (0, i)),
        ],
        out_specs=pl.BlockSpec((T, bv), lambda i: (0, i)),
        out_shape=jax.ShapeDtypeStruct((T, Vm), jnp.float32),
        scratch_shapes=[pltpu.VMEM((T, D), jnp.float32)],
        interpret=_I,
    )(x2, scale, lm_w, lm_b2)


def kernel(idx, emb_table, rms1_scale, in_proj_w, in_proj_b, out_proj_w,
           out_proj_b, router_w, router_b, W1, b1, W2, b2, rms_final_scale,
           lm_w, lm_b):
    B, T = idx.shape
    V, D = emb_table.shape
    half = T // 2

    HH = HD // 2
    theta = 1.0 / (10000.0 ** (jnp.arange(0, HD, 2, dtype=jnp.float32) / HD))
    ang = jnp.arange(T, dtype=jnp.float32)[:, None] * theta[None, :]
    c32 = jnp.cos(ang)
    s32 = jnp.sin(ang)
    c_idx = jnp.arange(D)
    jmap = (c_idx % HD) // 2
    ec = (jnp.arange(HH)[:, None] == jmap[None, :]).astype(jnp.float32)
    sgn = jnp.where(c_idx % 2 == 0, 1.0, -1.0)
    es = ec * sgn[None, :]

    Wq, Wk, Wv = in_proj_w[:D], in_proj_w[D:2 * D], in_proj_w[2 * D:]
    bq, bk, bv_ = in_proj_b[:D], in_proj_b[D:2 * D], in_proj_b[2 * D:]
    swap = jnp.arange(D).reshape(D // 2, 2)[:, ::-1].reshape(D)

    def qk_pack(wq_t, wk_t):
        return jnp.concatenate(
            [wq_t.reshape(D, NH, 1, HD), wk_t.reshape(D, NH, 1, HD)],
            axis=2).transpose(1, 0, 2, 3).reshape(NH, D, 2 * HD)

    wqk4 = qk_pack(Wq.T, Wk.T)
    wqks4 = qk_pack(Wq.T[swap], Wk.T[swap])
    wvT3 = Wv.T.reshape(D, NH, HD).transpose(1, 0, 2)
    bqk3 = jnp.concatenate(
        [bq.reshape(NH, 1, HD), bk.reshape(NH, 1, HD)], axis=-1)
    bv3 = bv_.reshape(NH, 1, HD)
    woT3 = out_proj_w.T.reshape(NH, HD, D)

    x = _emb_gather(emb_table, idx.reshape(T))
    q3, k3, v3 = _qkv(x, rms1_scale, c32, s32, ec, es,
                      wqk4, wqks4, wvT3, bqk3, bv3)
    ao3 = _attention(q3, k3, v3, half)
    xa, g = _postattn(ao3, woT3, out_proj_b, rms1_scale, router_w,
                      router_b.reshape(1, -1))
    x2 = _moe(xa, g, rms1_scale, W1, b1, W2, b2)
    logits = _lm_head(x2, rms_final_scale, lm_w, lm_b.reshape(1, -1))
    return logits.reshape(B, T, V - 1)


# speedup vs baseline: 1.0215x; 1.0215x over previous
import functools

import jax
import jax.numpy as jnp
from jax import lax
from jax.experimental import pallas as pl
from jax.experimental.pallas import tpu as pltpu
from jax.experimental.pallas import tpu_sc as plsc

NH = 12
HD = 64


def _rms(x, scale):
    return x * lax.rsqrt(jnp.mean(x * x, axis=-1, keepdims=True) + 1e-6) * scale


def _emb_gather(table, idx_flat):
    T = idx_flat.shape[0]
    V, D = table.shape
    NC, NS = 2, 16
    NW = NC * NS
    b_per_w = T // NW
    mesh = plsc.VectorSubcoreMesh(core_axis_name="c", subcore_axis_name="s")

    @functools.partial(
        pl.kernel, mesh=mesh,
        out_type=jax.ShapeDtypeStruct((T, D), jnp.float32),
        scratch_types=[
            pltpu.VMEM((b_per_w,), jnp.int32),
            pltpu.VMEM((b_per_w, D), jnp.float32),
            pltpu.SemaphoreType.DMA,
        ],
    )
    def gk(table_hbm, idx_hbm, out_hbm, idx_v, rows_v, sem):
        wid = lax.axis_index("s") * NC + lax.axis_index("c")
        base = wid * b_per_w
        pltpu.sync_copy(idx_hbm.at[pl.ds(base, b_per_w)], idx_v)
        pltpu.async_copy(table_hbm.at[idx_v], rows_v, sem).wait()
        pltpu.sync_copy(rows_v, out_hbm.at[pl.ds(base, b_per_w)])

    return gk(table, idx_flat)


def _qkv_body(x_ref, sc_ref, c32_ref, s32_ref, ec_ref, es_ref,
              wq_ref, wqs_ref, wk_ref, wks_ref,
              wv_ref, bq_ref, bk_ref, bv_ref, q_ref, k_ref, v_ref,
              c_s, s_s, *, blk):
    h = pl.program_id(1)
    dn0 = (((1,), (0,)), ((), ()))

    @pl.when(h == 0)
    def _():
        c_s[...] = lax.dot_general(c32_ref[...], ec_ref[...], dn0,
                                   preferred_element_type=jnp.float32)
        s_s[...] = lax.dot_general(s32_ref[...], es_ref[...], dn0,
                                   preferred_element_type=jnp.float32)

    xn = _rms(x_ref[...], sc_ref[...])
    a = xn * c_s[...]
    b = xn * s_s[...]
    dn = (((1,), (1,)), ((), ()))

    def mm(u, w_ref):
        return lax.dot_general(u, w_ref[0], dn,
                               preferred_element_type=jnp.float32)

    q_ref[0] = mm(a, wq_ref) + mm(b, wqs_ref) + bq_ref[0]
    k_ref[0] = mm(a, wk_ref) + mm(b, wks_ref) + bk_ref[0]
    v_ref[0] = mm(xn, wv_ref) + bv_ref[0]


def _qkv(x, scale, c32, s32, ec, es, wq3, wqs3, wk3, wks3, wv3, bq3, bk3, bv3,
         blk=512):
    T, D = x.shape
    HH = c32.shape[1]
    out = jax.ShapeDtypeStruct((NH, T, HD), jnp.float32)
    wspec = pl.BlockSpec((1, HD, D), lambda i, h: (h, 0, 0))
    bspec = pl.BlockSpec((1, 1, HD), lambda i, h: (h, 0, 0))
    return pl.pallas_call(
        functools.partial(_qkv_body, blk=blk),
        grid=(T // blk, NH),
        in_specs=[
            pl.BlockSpec((blk, D), lambda i, h: (i, 0)),
            pl.BlockSpec((D,), lambda i, h: (0,)),
            pl.BlockSpec((blk, HH), lambda i, h: (i, 0)),
            pl.BlockSpec((blk, HH), lambda i, h: (i, 0)),
            pl.BlockSpec((HH, D), lambda i, h: (0, 0)),
            pl.BlockSpec((HH, D), lambda i, h: (0, 0)),
            wspec, wspec, wspec, wspec, wspec,
            bspec, bspec, bspec,
        ],
        out_specs=[pl.BlockSpec((1, blk, HD), lambda i, h: (h, i, 0))] * 3,
        out_shape=[out, out, out],
        scratch_shapes=[pltpu.VMEM((blk, D), jnp.float32),
                        pltpu.VMEM((blk, D), jnp.float32)],
    )(x, scale, c32, s32, ec, es, wq3, wqs3, wk3, wks3, wv3, bq3, bk3, bv3)


def _attn_body(q_ref, k_ref, v_ref, o_ref, *, half, bq, win):
    qb = pl.program_id(1)
    q = q_ref[0]
    ntile = half // bq
    start = jnp.maximum(qb - ntile, 0) * bq
    kw = k_ref[0, pl.ds(start, win), :]
    vw = v_ref[0, pl.ds(start, win), :]
    dn = (((1,), (1,)), ((), ()))
    s = lax.dot_general(q, kw, dn, preferred_element_type=jnp.float32) * (HD ** -0.5)
    rows = qb * bq + lax.broadcasted_iota(jnp.int32, (bq, win), 0)
    cols = start + lax.broadcasted_iota(jnp.int32, (bq, win), 1)
    bad = (cols > rows) | (cols <= rows - half)
    s = jnp.where(bad, -jnp.inf, s)
    m = jnp.max(s, axis=-1, keepdims=True)
    p = jnp.exp(s - m)
    denom = jnp.sum(p, axis=-1, keepdims=True)
    o = lax.dot_general(p, vw, (((1,), (0,)), ((), ())),
                        preferred_element_type=jnp.float32)
    o_ref[0] = o / denom


def _attention(q3, k3, v3, half, bq=256):
    NHl, T, HDl = q3.shape
    win = half + bq
    body = functools.partial(_attn_body, half=half, bq=bq, win=win)
    return pl.pallas_call(
        body,
        grid=(NHl, T // bq),
        in_specs=[
            pl.BlockSpec((1, bq, HDl), lambda h, i: (h, i, 0)),
            pl.BlockSpec((1, T, HDl), lambda h, i: (h, 0, 0)),
            pl.BlockSpec((1, T, HDl), lambda h, i: (h, 0, 0)),
        ],
        out_specs=pl.BlockSpec((1, bq, HDl), lambda h, i: (h, i, 0)),
        out_shape=jax.ShapeDtypeStruct((NHl, T, HDl), jnp.float32),
    )(q3, k3, v3)


def _postattn_body(ao_ref, wo_ref, bo_ref, sc_ref, rw_ref, rb_ref,
                   xa_ref, g_ref):
    E = rw_ref.shape[0]
    dn = (((1,), (1,)), ((), ()))
    xa = bo_ref[...] + jnp.zeros((ao_ref.shape[1], wo_ref.shape[2]),
                                 jnp.float32)
    for h in range(ao_ref.shape[0]):
        xa = xa + lax.dot_general(ao_ref[h], wo_ref[h],
                                  (((1,), (0,)), ((), ())),
                                  preferred_element_type=jnp.float32)
    xa_ref[...] = xa
    hn = _rms(xa, sc_ref[...])
    lg = (lax.dot_general(hn, rw_ref[...], dn,
                          preferred_element_type=jnp.float32) + rb_ref[0])
    col = lax.broadcasted_iota(jnp.int32, lg.shape, 1)
    m1 = jnp.max(lg, axis=-1, keepdims=True)
    i1 = jnp.min(jnp.where(lg == m1, col, E), axis=-1, keepdims=True)
    lg2 = jnp.where(col == i1, -jnp.inf, lg)
    m2 = jnp.max(lg2, axis=-1, keepdims=True)
    i2 = jnp.min(jnp.where(lg2 == m2, col, E), axis=-1, keepdims=True)
    keep = (col == i1) | (col == i2)
    sp = jnp.where(keep, lg, -jnp.inf)
    p = jnp.exp(sp - m1)
    g_ref[...] = p / jnp.sum(p, axis=-1, keepdims=True)


def _postattn(ao3, woT3, bo, scale, rw, rb2, blk=256):
    _, T, _ = ao3.shape
    D = woT3.shape[2]
    E = rw.shape[0]
    return pl.pallas_call(
        _postattn_body,
        grid=(T // blk,),
        in_specs=[
            pl.BlockSpec((NH, blk, HD), lambda i: (0, i, 0)),
            pl.BlockSpec((NH, HD, D), lambda i: (0, 0, 0)),
            pl.BlockSpec((D,), lambda i: (0,)),
            pl.BlockSpec((D,), lambda i: (0,)),
            pl.BlockSpec((E, D), lambda i: (0, 0)),
            pl.BlockSpec((1, E), lambda i: (0, 0)),
        ],
        out_specs=[
            pl.BlockSpec((blk, D), lambda i: (i, 0)),
            pl.BlockSpec((blk, E), lambda i: (i, 0)),
        ],
        out_shape=[
            jax.ShapeDtypeStruct((T, D), jnp.float32),
            jax.ShapeDtypeStruct((T, E), jnp.float32),
        ],
    )(ao3, woT3, bo, scale, rw, rb2)


def _moe_body(xa_ref, g_ref, sc_ref, w1_ref, b1_ref, w2_ref, b2_ref, o_ref,
              hn_s):
    e = pl.program_id(0)
    f = pl.program_id(1)
    first = (e == 0) & (f == 0)

    @pl.when(first)
    def _():
        hn_s[...] = _rms(xa_ref[...], sc_ref[...])
        o_ref[...] = xa_ref[...]

    h = lax.dot_general(hn_s[...], w1_ref[0], (((1,), (0,)), ((), ())),
                        preferred_element_type=jnp.float32) + b1_ref[0, 0]
    h = h * jax.nn.sigmoid(h)
    part = lax.dot_general(h, w2_ref[0], (((1,), (0,)), ((), ())),
                           preferred_element_type=jnp.float32)
    g = g_ref[...]
    col = lax.broadcasted_iota(jnp.int32, g.shape, 1)
    ge = jnp.sum(jnp.where(col == e, g, 0.0), axis=-1, keepdims=True)
    contrib = part * ge
    contrib = contrib + jnp.where(f == 0, 1.0, 0.0) * (ge * b2_ref[0, 0])
    o_ref[...] = o_ref[...] + contrib


def _moe(xa, g, scale, W1, b1, W2, b2, bf=768):
    T, D = xa.shape
    E, _, FF = W1.shape
    return pl.pallas_call(
        _moe_body,
        grid=(E, FF // bf),
        in_specs=[
            pl.BlockSpec((T, D), lambda e, f: (0, 0)),
            pl.BlockSpec((T, E), lambda e, f: (0, 0)),
            pl.BlockSpec((D,), lambda e, f: (0,)),
            pl.BlockSpec((1, D, bf), lambda e, f: (e, 0, f)),
            pl.BlockSpec((1, 1, bf), lambda e, f: (e, 0, f)),
            pl.BlockSpec((1, bf, D), lambda e, f: (e, f, 0)),
            pl.BlockSpec((1, 1, D), lambda e, f: (e, 0, 0)),
        ],
        out_specs=pl.BlockSpec((T, D), lambda e, f: (0, 0)),
        out_shape=jax.ShapeDtypeStruct((T, D), jnp.float32),
        scratch_shapes=[pltpu.VMEM((T, D), jnp.float32)],
    )(xa, g, scale, W1, b1.reshape(E, 1, FF), W2, b2.reshape(E, 1, D))


def _lm_body(x_ref, sc_ref, w_ref, b_ref, o_ref, xn_s):
    @pl.when(pl.program_id(0) == 0)
    def _():
        xn_s[...] = _rms(x_ref[...], sc_ref[...])

    o_ref[...] = (lax.dot_general(xn_s[...], w_ref[...],
                                  (((1,), (1,)), ((), ())),
                                  preferred_element_type=jnp.float32)
                  + b_ref[0])


def _lm_head(x2, scale, lm_w, lm_b2, bv=1024):
    T, D = x2.shape
    Vm = lm_w.shape[0]
    return pl.pallas_call(
        _lm_body,
        grid=(pl.cdiv(Vm, bv),),
        in_specs=[
            pl.BlockSpec((T, D), lambda i: (0, 0)),
            pl.BlockSpec((D,), lambda i: (0,)),
            pl.BlockSpec((bv, D), lambda i: (i, 0)),
            pl.BlockSpec((1, bv), lambda i: (0, i)),
        ],
        out_specs=pl.BlockSpec((T, bv), lambda i: (0, i)),
        out_shape=jax.ShapeDtypeStruct((T, Vm), jnp.float32),
        scratch_shapes=[pltpu.VMEM((T, D), jnp.float32)],
    )(x2, scale, lm_w, lm_b2)


def kernel(idx, emb_table, rms1_scale, in_proj_w, in_proj_b, out_proj_w,
           out_proj_b, router_w, router_b, W1, b1, W2, b2, rms_final_scale,
           lm_w, lm_b):
    B, T = idx.shape
    V, D = emb_table.shape
    half = T // 2

    HH = HD // 2
    theta = 1.0 / (10000.0 ** (jnp.arange(0, HD, 2, dtype=jnp.float32) / HD))
    ang = jnp.arange(T, dtype=jnp.float32)[:, None] * theta[None, :]
    c32 = jnp.cos(ang)
    s32 = jnp.sin(ang)
    c_idx = jnp.arange(D)
    jmap = (c_idx % HD) // 2
    ec = (jnp.arange(HH)[:, None] == jmap[None, :]).astype(jnp.float32)
    sgn = jnp.where(c_idx % 2 == 0, 1.0, -1.0)
    es = ec * sgn[None, :]

    Wq, Wk, Wv = in_proj_w[:D], in_proj_w[D:2 * D], in_proj_w[2 * D:]
    bq, bk, bv_ = in_proj_b[:D], in_proj_b[D:2 * D], in_proj_b[2 * D:]
    swap = jnp.arange(D).reshape(D // 2, 2)[:, ::-1].reshape(D)

    wq3 = Wq.reshape(NH, HD, D)
    wqs3 = Wq[:, swap].reshape(NH, HD, D)
    wk3 = Wk.reshape(NH, HD, D)
    wks3 = Wk[:, swap].reshape(NH, HD, D)
    wv3 = Wv.reshape(NH, HD, D)
    bq3 = bq.reshape(NH, 1, HD)
    bk3 = bk.reshape(NH, 1, HD)
    bv3 = bv_.reshape(NH, 1, HD)
    woT3 = out_proj_w.T.reshape(NH, HD, D)

    x = _emb_gather(emb_table, idx.reshape(T))
    q3, k3, v3 = _qkv(x, rms1_scale, c32, s32, ec, es,
                      wq3, wqs3, wk3, wks3, wv3, bq3, bk3, bv3)
    ao3 = _attention(q3, k3, v3, half)
    xa, g = _postattn(ao3, woT3, out_proj_b, rms1_scale, router_w,
                      router_b.reshape(1, -1))
    x2 = _moe(xa, g, rms1_scale, W1, b1, W2, b2)
    logits = _lm_head(x2, rms_final_scale, lm_w, lm_b.reshape(1, -1))
    return logits.reshape(B, T, V - 1)


# speedup vs baseline: 1.0336x; 1.0118x over previous
import functools

import jax
import jax.numpy as jnp
from jax import lax
from jax.experimental import pallas as pl
from jax.experimental.pallas import tpu as pltpu
from jax.experimental.pallas import tpu_sc as plsc

NH = 12
HD = 64


def _rms(x, scale):
    return x * lax.rsqrt(jnp.mean(x * x, axis=-1, keepdims=True) + 1e-6) * scale


def _emb_gather(table, idx_flat):
    T = idx_flat.shape[0]
    V, D = table.shape
    NC, NS = 2, 16
    NW = NC * NS
    b_per_w = T // NW
    mesh = plsc.VectorSubcoreMesh(core_axis_name="c", subcore_axis_name="s")

    @functools.partial(
        pl.kernel, mesh=mesh,
        out_type=jax.ShapeDtypeStruct((T, D), jnp.float32),
        scratch_types=[
            pltpu.VMEM((b_per_w,), jnp.int32),
            pltpu.VMEM((b_per_w, D), jnp.float32),
            pltpu.SemaphoreType.DMA,
        ],
    )
    def gk(table_hbm, idx_hbm, out_hbm, idx_v, rows_v, sem):
        wid = lax.axis_index("s") * NC + lax.axis_index("c")
        base = wid * b_per_w
        pltpu.sync_copy(idx_hbm.at[pl.ds(base, b_per_w)], idx_v)
        pltpu.async_copy(table_hbm.at[idx_v], rows_v, sem).wait()
        pltpu.sync_copy(rows_v, out_hbm.at[pl.ds(base, b_per_w)])

    return gk(table, idx_flat)


def _qkv_body(x_ref, sc_ref, c32_ref, s32_ref, ec_ref, es_ref,
              wq_ref, wqs_ref, wk_ref, wks_ref,
              wv_ref, bq_ref, bk_ref, bv_ref, q_ref, k_ref, v_ref,
              c_s, s_s, *, blk):
    h = pl.program_id(1)
    dn0 = (((1,), (0,)), ((), ()))

    @pl.when(h == 0)
    def _():
        c_s[...] = lax.dot_general(c32_ref[...], ec_ref[...], dn0,
                                   preferred_element_type=jnp.float32)
        s_s[...] = lax.dot_general(s32_ref[...], es_ref[...], dn0,
                                   preferred_element_type=jnp.float32)

    xn = _rms(x_ref[...], sc_ref[...])
    a = xn * c_s[...]
    b = xn * s_s[...]
    dn = (((1,), (1,)), ((), ()))

    def mm(u, w_ref):
        return lax.dot_general(u, w_ref[0], dn,
                               preferred_element_type=jnp.float32)

    q_ref[0] = mm(a, wq_ref) + mm(b, wqs_ref) + bq_ref[0]
    k_ref[0] = mm(a, wk_ref) + mm(b, wks_ref) + bk_ref[0]
    v_ref[0] = mm(xn, wv_ref) + bv_ref[0]


def _qkv(x, scale, c32, s32, ec, es, wq3, wqs3, wk3, wks3, wv3, bq3, bk3, bv3,
         blk=512):
    T, D = x.shape
    HH = c32.shape[1]
    out = jax.ShapeDtypeStruct((NH, T, HD), jnp.float32)
    wspec = pl.BlockSpec((1, HD, D), lambda i, h: (h, 0, 0))
    bspec = pl.BlockSpec((1, 1, HD), lambda i, h: (h, 0, 0))
    return pl.pallas_call(
        functools.partial(_qkv_body, blk=blk),
        grid=(T // blk, NH),
        in_specs=[
            pl.BlockSpec((blk, D), lambda i, h: (i, 0)),
            pl.BlockSpec((D,), lambda i, h: (0,)),
            pl.BlockSpec((blk, HH), lambda i, h: (i, 0)),
            pl.BlockSpec((blk, HH), lambda i, h: (i, 0)),
            pl.BlockSpec((HH, D), lambda i, h: (0, 0)),
            pl.BlockSpec((HH, D), lambda i, h: (0, 0)),
            wspec, wspec, wspec, wspec, wspec,
            bspec, bspec, bspec,
        ],
        out_specs=[pl.BlockSpec((1, blk, HD), lambda i, h: (h, i, 0))] * 3,
        out_shape=[out, out, out],
        scratch_shapes=[pltpu.VMEM((blk, D), jnp.float32),
                        pltpu.VMEM((blk, D), jnp.float32)],
    )(x, scale, c32, s32, ec, es, wq3, wqs3, wk3, wks3, wv3, bq3, bk3, bv3)


def _attn_body(q_ref, k_ref, v_ref, o_ref, *, half, bq, win):
    qb = pl.program_id(1)
    q = q_ref[0]
    ntile = half // bq
    start = jnp.maximum(qb - ntile, 0) * bq
    kw = k_ref[0, pl.ds(start, win), :]
    vw = v_ref[0, pl.ds(start, win), :]
    dn = (((1,), (1,)), ((), ()))
    s = lax.dot_general(q, kw, dn, preferred_element_type=jnp.float32) * (HD ** -0.5)
    rows = qb * bq + lax.broadcasted_iota(jnp.int32, (bq, win), 0)
    cols = start + lax.broadcasted_iota(jnp.int32, (bq, win), 1)
    bad = (cols > rows) | (cols <= rows - half)
    s = jnp.where(bad, -jnp.inf, s)
    m = jnp.max(s, axis=-1, keepdims=True)
    p = jnp.exp(s - m)
    denom = jnp.sum(p, axis=-1, keepdims=True)
    o = lax.dot_general(p, vw, (((1,), (0,)), ((), ())),
                        preferred_element_type=jnp.float32)
    o_ref[0] = o / denom


def _attention(q3, k3, v3, half, bq=512):
    NHl, T, HDl = q3.shape
    win = half + bq
    body = functools.partial(_attn_body, half=half, bq=bq, win=win)
    return pl.pallas_call(
        body,
        grid=(NHl, T // bq),
        in_specs=[
            pl.BlockSpec((1, bq, HDl), lambda h, i: (h, i, 0)),
            pl.BlockSpec((1, T, HDl), lambda h, i: (h, 0, 0)),
            pl.BlockSpec((1, T, HDl), lambda h, i: (h, 0, 0)),
        ],
        out_specs=pl.BlockSpec((1, bq, HDl), lambda h, i: (h, i, 0)),
        out_shape=jax.ShapeDtypeStruct((NHl, T, HDl), jnp.float32),
    )(q3, k3, v3)


def _postattn_body(ao_ref, wo_ref, bo_ref, sc_ref, rw_ref, rb_ref,
                   xa_ref, g_ref):
    E = rw_ref.shape[0]
    dn = (((1,), (1,)), ((), ()))
    xa = bo_ref[...] + jnp.zeros((ao_ref.shape[1], wo_ref.shape[2]),
                                 jnp.float32)
    for h in range(ao_ref.shape[0]):
        xa = xa + lax.dot_general(ao_ref[h], wo_ref[h],
                                  (((1,), (0,)), ((), ())),
                                  preferred_element_type=jnp.float32)
    xa_ref[...] = xa
    hn = _rms(xa, sc_ref[...])
    lg = (lax.dot_general(hn, rw_ref[...], dn,
                          preferred_element_type=jnp.float32) + rb_ref[0])
    col = lax.broadcasted_iota(jnp.int32, lg.shape, 1)
    m1 = jnp.max(lg, axis=-1, keepdims=True)
    i1 = jnp.min(jnp.where(lg == m1, col, E), axis=-1, keepdims=True)
    lg2 = jnp.where(col == i1, -jnp.inf, lg)
    m2 = jnp.max(lg2, axis=-1, keepdims=True)
    i2 = jnp.min(jnp.where(lg2 == m2, col, E), axis=-1, keepdims=True)
    keep = (col == i1) | (col == i2)
    sp = jnp.where(keep, lg, -jnp.inf)
    p = jnp.exp(sp - m1)
    g_ref[...] = p / jnp.sum(p, axis=-1, keepdims=True)


def _postattn(ao3, woT3, bo, scale, rw, rb2, blk=256):
    _, T, _ = ao3.shape
    D = woT3.shape[2]
    E = rw.shape[0]
    return pl.pallas_call(
        _postattn_body,
        grid=(T // blk,),
        in_specs=[
            pl.BlockSpec((NH, blk, HD), lambda i: (0, i, 0)),
            pl.BlockSpec((NH, HD, D), lambda i: (0, 0, 0)),
            pl.BlockSpec((D,), lambda i: (0,)),
            pl.BlockSpec((D,), lambda i: (0,)),
            pl.BlockSpec((E, D), lambda i: (0, 0)),
            pl.BlockSpec((1, E), lambda i: (0, 0)),
        ],
        out_specs=[
            pl.BlockSpec((blk, D), lambda i: (i, 0)),
            pl.BlockSpec((blk, E), lambda i: (i, 0)),
        ],
        out_shape=[
            jax.ShapeDtypeStruct((T, D), jnp.float32),
            jax.ShapeDtypeStruct((T, E), jnp.float32),
        ],
    )(ao3, woT3, bo, scale, rw, rb2)


def _moe_body(xa_ref, g_ref, sc_ref, w1_ref, b1_ref, w2_ref, b2_ref, o_ref,
              hn_s):
    e = pl.program_id(0)
    f = pl.program_id(1)
    first = (e == 0) & (f == 0)

    @pl.when(first)
    def _():
        hn_s[...] = _rms(xa_ref[...], sc_ref[...])
        o_ref[...] = xa_ref[...]

    h = lax.dot_general(hn_s[...], w1_ref[0], (((1,), (0,)), ((), ())),
                        preferred_element_type=jnp.float32) + b1_ref[0, 0]
    h = h * jax.nn.sigmoid(h)
    part = lax.dot_general(h, w2_ref[0], (((1,), (0,)), ((), ())),
                           preferred_element_type=jnp.float32)
    g = g_ref[...]
    col = lax.broadcasted_iota(jnp.int32, g.shape, 1)
    ge = jnp.sum(jnp.where(col == e, g, 0.0), axis=-1, keepdims=True)
    contrib = part * ge
    contrib = contrib + jnp.where(f == 0, 1.0, 0.0) * (ge * b2_ref[0, 0])
    o_ref[...] = o_ref[...] + contrib


def _moe(xa, g, scale, W1, b1, W2, b2, bf=1024):
    T, D = xa.shape
    E, _, FF = W1.shape
    return pl.pallas_call(
        _moe_body,
        grid=(E, FF // bf),
        in_specs=[
            pl.BlockSpec((T, D), lambda e, f: (0, 0)),
            pl.BlockSpec((T, E), lambda e, f: (0, 0)),
            pl.BlockSpec((D,), lambda e, f: (0,)),
            pl.BlockSpec((1, D, bf), lambda e, f: (e, 0, f)),
            pl.BlockSpec((1, 1, bf), lambda e, f: (e, 0, f)),
            pl.BlockSpec((1, bf, D), lambda e, f: (e, f, 0)),
            pl.BlockSpec((1, 1, D), lambda e, f: (e, 0, 0)),
        ],
        out_specs=pl.BlockSpec((T, D), lambda e, f: (0, 0)),
        out_shape=jax.ShapeDtypeStruct((T, D), jnp.float32),
        scratch_shapes=[pltpu.VMEM((T, D), jnp.float32)],
    )(xa, g, scale, W1, b1.reshape(E, 1, FF), W2, b2.reshape(E, 1, D))


def _lm_body(x_ref, sc_ref, w_ref, b_ref, o_ref, xn_s):
    @pl.when(pl.program_id(0) == 0)
    def _():
        xn_s[...] = _rms(x_ref[...], sc_ref[...])

    o_ref[...] = (lax.dot_general(xn_s[...], w_ref[...],
                                  (((1,), (1,)), ((), ())),
                                  preferred_element_type=jnp.float32)
                  + b_ref[0])


def _lm_head(x2, scale, lm_w, lm_b2, bv=1024):
    T, D = x2.shape
    Vm = lm_w.shape[0]
    return pl.pallas_call(
        _lm_body,
        grid=(pl.cdiv(Vm, bv),),
        in_specs=[
            pl.BlockSpec((T, D), lambda i: (0, 0)),
            pl.BlockSpec((D,), lambda i: (0,)),
            pl.BlockSpec((bv, D), lambda i: (i, 0)),
            pl.BlockSpec((1, bv), lambda i: (0, i)),
        ],
        out_specs=pl.BlockSpec((T, bv), lambda i: (0, i)),
        out_shape=jax.ShapeDtypeStruct((T, Vm), jnp.float32),
        scratch_shapes=[pltpu.VMEM((T, D), jnp.float32)],
    )(x2, scale, lm_w, lm_b2)


def kernel(idx, emb_table, rms1_scale, in_proj_w, in_proj_b, out_proj_w,
           out_proj_b, router_w, router_b, W1, b1, W2, b2, rms_final_scale,
           lm_w, lm_b):
    B, T = idx.shape
    V, D = emb_table.shape
    half = T // 2

    HH = HD // 2
    theta = 1.0 / (10000.0 ** (jnp.arange(0, HD, 2, dtype=jnp.float32) / HD))
    ang = jnp.arange(T, dtype=jnp.float32)[:, None] * theta[None, :]
    c32 = jnp.cos(ang)
    s32 = jnp.sin(ang)
    c_idx = jnp.arange(D)
    jmap = (c_idx % HD) // 2
    ec = (jnp.arange(HH)[:, None] == jmap[None, :]).astype(jnp.float32)
    sgn = jnp.where(c_idx % 2 == 0, 1.0, -1.0)
    es = ec * sgn[None, :]

    Wq, Wk, Wv = in_proj_w[:D], in_proj_w[D:2 * D], in_proj_w[2 * D:]
    bq, bk, bv_ = in_proj_b[:D], in_proj_b[D:2 * D], in_proj_b[2 * D:]
    swap = jnp.arange(D).reshape(D // 2, 2)[:, ::-1].reshape(D)

    wq3 = Wq.reshape(NH, HD, D)
    wqs3 = Wq[:, swap].reshape(NH, HD, D)
    wk3 = Wk.reshape(NH, HD, D)
    wks3 = Wk[:, swap].reshape(NH, HD, D)
    wv3 = Wv.reshape(NH, HD, D)
    bq3 = bq.reshape(NH, 1, HD)
    bk3 = bk.reshape(NH, 1, HD)
    bv3 = bv_.reshape(NH, 1, HD)
    woT3 = out_proj_w.T.reshape(NH, HD, D)

    x = _emb_gather(emb_table, idx.reshape(T))
    q3, k3, v3 = _qkv(x, rms1_scale, c32, s32, ec, es,
                      wq3, wqs3, wk3, wks3, wv3, bq3, bk3, bv3)
    ao3 = _attention(q3, k3, v3, half)
    xa, g = _postattn(ao3, woT3, out_proj_b, rms1_scale, router_w,
                      router_b.reshape(1, -1))
    x2 = _moe(xa, g, rms1_scale, W1, b1, W2, b2)
    logits = _lm_head(x2, rms_final_scale, lm_w, lm_b.reshape(1, -1))
    return logits.reshape(B, T, V - 1)
